# baseline (speedup 1.0000x reference)
import functools

import jax
import jax.numpy as jnp
from jax import lax
from jax.experimental import pallas as pl
from jax.experimental.pallas import tpu as pltpu
from jax.experimental.pallas import tpu_sc as plsc

_NC = 2
_NS = 16
_NW = _NC * _NS
_L = 16


def _dedr_body(ev9_ref, wt9_ref, wt_ref, b_ref, d0_ref, d1_ref, d2_ref):
    ev9 = ev9_ref[...]
    wt9 = wt9_ref[...]
    wt = wt_ref[...]
    h = b_ref[...] + lax.dot_general(
        wt9, ev9, dimension_numbers=(((1,), (0,)), ((), ())),
        precision=lax.Precision.DEFAULT,
        preferred_element_type=jnp.float32)
    t = jnp.tanh(h)
    g = 1.0 - t * t
    d = lax.dot_general(
        wt, g, dimension_numbers=(((0,), (0,)), ((), ())),
        precision=lax.Precision.DEFAULT,
        preferred_element_type=jnp.float32)
    eb = ev9.shape[1]
    base = pl.program_id(0) * eb
    d0_ref[pl.ds(base, eb)] = d[0]
    d1_ref[pl.ds(base, eb)] = d[1]
    d2_ref[pl.ds(base, eb)] = d[2]


def _dedr_tc(ev9, Wt9, Wt, b2):
    n_edges = ev9.shape[1]
    h = Wt.shape[0]
    eb = 12800
    assert n_edges % eb == 0
    out1d = jax.ShapeDtypeStruct((n_edges,), jnp.float32)
    return pl.pallas_call(
        _dedr_body,
        grid=(n_edges // eb,),
        in_specs=[
            pl.BlockSpec((9, eb), lambda i: (0, i)),
            pl.BlockSpec((h, 9), lambda i: (0, 0)),
            pl.BlockSpec((h, 3), lambda i: (0, 0)),
            pl.BlockSpec((h, 1), lambda i: (0, 0)),
        ],
        out_specs=[pl.BlockSpec((n_edges,), lambda i: (0,))] * 3,
        out_shape=[out1d, out1d, out1d],
    )(ev9, Wt9, Wt, b2)


def _make_scatter(n_edges, aw):
    epw = n_edges // _NW
    assert n_edges % (_NW * _L) == 0 and epw % 8 == 0
    mesh = plsc.VectorSubcoreMesh(core_axis_name="c", subcore_axis_name="s")

    @functools.partial(
        pl.kernel,
        out_type=jax.ShapeDtypeStruct((_NW * aw,), jnp.float32),
        mesh=mesh,
        scratch_types=[
            pltpu.VMEM((epw,), jnp.float32),
            pltpu.VMEM((epw,), jnp.float32),
            pltpu.VMEM((epw,), jnp.float32),
            pltpu.VMEM((epw,), jnp.int32),
            pltpu.VMEM((epw,), jnp.int32),
            pltpu.VMEM((aw,), jnp.float32),
            pltpu.SemaphoreType.DMA,
        ],
        compiler_params=pltpu.CompilerParams(needs_layout_passes=False),
    )
    def scatter_k(d0_h, d1_h, d2_h, src_h, dst_h, out_h,
                  d0_v, d1_v, d2_v, src_v, dst_v, acc_v, sem):
        wid = lax.axis_index("s") * _NC + lax.axis_index("c")
        base = wid * epw
        d_v = (d0_v, d1_v, d2_v)
        cps = [pltpu.async_copy(h_ref.at[pl.ds(base, epw)], v_ref, sem)
               for h_ref, v_ref in ((d0_h, d0_v), (d1_h, d1_v), (d2_h, d2_v),
                                    (src_h, src_v), (dst_h, dst_v))]

        zeros = jnp.zeros((_L,), jnp.float32)
        zu = 16

        def zbody(i, carry):
            for u in range(zu):
                acc_v[pl.ds(i * (_L * zu) + u * _L, _L)] = zeros
            return carry

        lax.fori_loop(0, aw // (_L * zu), zbody, 0)
        for cp in cps:
            cp.wait()

        su = 25
        assert (epw // _L) % su == 0

        def sbody(g, carry):
            for u in range(su):
                off = g * (_L * su) + u * _L
                si = src_v[pl.ds(off, _L)] * 3
                di = dst_v[pl.ds(off, _L)] * 3
                for c in range(3):
                    v = d_v[c][pl.ds(off, _L)]
                    plsc.addupdate_scatter(acc_v, [si + c], v)
                    plsc.addupdate_scatter(acc_v, [di + c], -v)
            return carry

        lax.fori_loop(0, epw // (_L * su), sbody, 0)
        pltpu.sync_copy(acc_v, out_h.at[pl.ds(wid * aw, aw)])

    return scatter_k


def _make_reduce(aw):
    cw = aw // _NW
    assert cw % _L == 0 and cw % 8 == 0
    mesh = plsc.VectorSubcoreMesh(core_axis_name="c", subcore_axis_name="s")

    @functools.partial(
        pl.kernel,
        out_type=jax.ShapeDtypeStruct((aw,), jnp.float32),
        mesh=mesh,
        scratch_types=[
            pltpu.VMEM((_NW, cw), jnp.float32),
            pltpu.VMEM((cw,), jnp.float32),
            pltpu.SemaphoreType.DMA,
        ],
        compiler_params=pltpu.CompilerParams(needs_layout_passes=False),
    )
    def reduce_k(part_h, out_h, blk_v, res_v, sem):
        wid = lax.axis_index("s") * _NC + lax.axis_index("c")
        base = wid * cw
        cps = [pltpu.async_copy(part_h.at[pl.ds(r * aw + base, cw)],
                                blk_v.at[r], sem)
               for r in range(_NW)]
        for cp in cps:
            cp.wait()

        def cbody(j, carry):
            off = j * _L
            s = blk_v[0, pl.ds(off, _L)]
            for r in range(1, _NW):
                s = s + blk_v[r, pl.ds(off, _L)]
            res_v[pl.ds(off, _L)] = s
            return carry

        lax.fori_loop(0, cw // _L, cbody, 0)
        pltpu.sync_copy(res_v, out_h.at[pl.ds(base, cw)])

    return reduce_k


def kernel(node_feature, edge_vec, edge_index, W, b):
    n_nodes = node_feature.shape[0]
    n_edges = edge_vec.shape[0]
    aw = -(-(3 * n_nodes) // (_NW * 128)) * (_NW * 128)

    cols = [edge_vec[:, k] for k in range(3)]
    his = [c.astype(jnp.bfloat16) for c in cols]
    los = [(c - h.astype(jnp.float32)).astype(jnp.bfloat16)
           for c, h in zip(cols, his)]
    ev9 = jnp.stack(his + los + his, axis=0)
    wt = W.T
    wt_hi = wt.astype(jnp.bfloat16)
    wt_lo = (wt - wt_hi.astype(jnp.float32)).astype(jnp.bfloat16)
    wt9 = jnp.concatenate([wt_hi, wt_hi, wt_lo], axis=1)
    d0, d1, d2 = _dedr_tc(ev9, wt9, wt, b.reshape(-1, 1))
    parts = _make_scatter(n_edges, aw)(
        d0, d1, d2, edge_index[0], edge_index[1])
    flat = _make_reduce(aw)(parts)
    return flat[: 3 * n_nodes].reshape(n_nodes, 3)


# speedup vs baseline: 1.3482x; 1.3482x over previous
import functools

import jax
import jax.numpy as jnp
from jax import lax
from jax.experimental import pallas as pl
from jax.experimental.pallas import tpu as pltpu
from jax.experimental.pallas import tpu_sc as plsc

_NC = 2
_NS = 16
_NW = _NC * _NS
_L = 16


def _dedr_body(ev9_ref, wt9_ref, wt_ref, b_ref, d0_ref, d1_ref, d2_ref):
    ev9 = ev9_ref[...]
    wt9 = wt9_ref[...]
    wt = wt_ref[...]
    h = b_ref[...] + lax.dot_general(
        wt9, ev9, dimension_numbers=(((1,), (0,)), ((), ())),
        precision=lax.Precision.DEFAULT,
        preferred_element_type=jnp.float32)
    t = jnp.tanh(h)
    g = 1.0 - t * t
    d = lax.dot_general(
        wt, g, dimension_numbers=(((0,), (0,)), ((), ())),
        precision=lax.Precision.DEFAULT,
        preferred_element_type=jnp.float32)
    eb = ev9.shape[1]
    base = pl.program_id(0) * eb
    d0_ref[pl.ds(base, eb)] = d[0]
    d1_ref[pl.ds(base, eb)] = d[1]
    d2_ref[pl.ds(base, eb)] = d[2]


def _dedr_tc(ev9, Wt9, Wt, b2):
    n_edges = ev9.shape[1]
    h = Wt.shape[0]
    eb = 12800
    assert n_edges % eb == 0
    out1d = jax.ShapeDtypeStruct((n_edges,), jnp.float32)
    return pl.pallas_call(
        _dedr_body,
        grid=(n_edges // eb,),
        in_specs=[
            pl.BlockSpec((9, eb), lambda i: (0, i)),
            pl.BlockSpec((h, 9), lambda i: (0, 0)),
            pl.BlockSpec((h, 3), lambda i: (0, 0)),
            pl.BlockSpec((h, 1), lambda i: (0, 0)),
        ],
        out_specs=[pl.BlockSpec((n_edges,), lambda i: (0,))] * 3,
        out_shape=[out1d, out1d, out1d],
    )(ev9, Wt9, Wt, b2)


def _make_scatter(n_edges, aw):
    epw = n_edges // _NW
    assert n_edges % (_NW * _L) == 0 and epw % 8 == 0
    mesh = plsc.VectorSubcoreMesh(core_axis_name="c", subcore_axis_name="s")

    @functools.partial(
        pl.kernel,
        out_type=jax.ShapeDtypeStruct((_NW * aw,), jnp.float32),
        mesh=mesh,
        scratch_types=[
            pltpu.VMEM((epw,), jnp.float32),
            pltpu.VMEM((epw,), jnp.float32),
            pltpu.VMEM((epw,), jnp.float32),
            pltpu.VMEM((epw,), jnp.int32),
            pltpu.VMEM((epw,), jnp.int32),
            pltpu.VMEM((aw,), jnp.float32),
            pltpu.SemaphoreType.DMA,
        ],
        compiler_params=pltpu.CompilerParams(needs_layout_passes=False),
    )
    def scatter_k(d0_h, d1_h, d2_h, src_h, dst_h, out_h,
                  d0_v, d1_v, d2_v, src_v, dst_v, acc_v, sem):
        wid = lax.axis_index("s") * _NC + lax.axis_index("c")
        base = wid * epw
        d_v = (d0_v, d1_v, d2_v)
        cps = [pltpu.async_copy(h_ref.at[pl.ds(base, epw)], v_ref, sem)
               for h_ref, v_ref in ((d0_h, d0_v), (d1_h, d1_v), (d2_h, d2_v),
                                    (src_h, src_v), (dst_h, dst_v))]

        zeros = jnp.zeros((_L,), jnp.float32)
        zu = 16

        def zbody(i, carry):
            for u in range(zu):
                acc_v[pl.ds(i * (_L * zu) + u * _L, _L)] = zeros
            return carry

        lax.fori_loop(0, aw // (_L * zu), zbody, 0)
        for cp in cps:
            cp.wait()

        su = 25
        assert (epw // _L) % su == 0

        def sbody(g, carry):
            for u in range(su):
                off = g * (_L * su) + u * _L
                si = src_v[pl.ds(off, _L)] * 3
                di = dst_v[pl.ds(off, _L)] * 3
                for c in range(3):
                    v = d_v[c][pl.ds(off, _L)]
                    plsc.addupdate_scatter(acc_v, [si + c], v)
                    plsc.addupdate_scatter(acc_v, [di + c], -v)
            return carry

        lax.fori_loop(0, epw // (_L * su), sbody, 0)
        pltpu.sync_copy(acc_v, out_h.at[pl.ds(wid * aw, aw)])

    return scatter_k


def _make_reduce(aw):
    cw = aw // _NW
    assert cw % _L == 0 and cw % 8 == 0
    mesh = plsc.VectorSubcoreMesh(core_axis_name="c", subcore_axis_name="s")

    @functools.partial(
        pl.kernel,
        out_type=jax.ShapeDtypeStruct((aw,), jnp.float32),
        mesh=mesh,
        scratch_types=[
            pltpu.VMEM((_NW, cw), jnp.float32),
            pltpu.VMEM((cw,), jnp.float32),
            pltpu.SemaphoreType.DMA,
        ],
        compiler_params=pltpu.CompilerParams(needs_layout_passes=False),
    )
    def reduce_k(part_h, out_h, blk_v, res_v, sem):
        wid = lax.axis_index("s") * _NC + lax.axis_index("c")
        base = wid * cw
        cps = [pltpu.async_copy(part_h.at[pl.ds(r * aw + base, cw)],
                                blk_v.at[r], sem)
               for r in range(_NW)]
        for cp in cps:
            cp.wait()

        def cbody(j, carry):
            off = j * _L
            s = blk_v[0, pl.ds(off, _L)]
            for r in range(1, _NW):
                s = s + blk_v[r, pl.ds(off, _L)]
            res_v[pl.ds(off, _L)] = s
            return carry

        lax.fori_loop(0, cw // _L, cbody, 0)
        pltpu.sync_copy(res_v, out_h.at[pl.ds(base, cw)])

    return reduce_k


def kernel(node_feature, edge_vec, edge_index, W, b):
    n_nodes = node_feature.shape[0]
    n_edges = edge_vec.shape[0]
    aw = -(-(3 * n_nodes) // (_NW * 128)) * (_NW * 128)

    evT = edge_vec.T
    ev_hi = evT.astype(jnp.bfloat16)
    ev_lo = (evT - ev_hi.astype(jnp.float32)).astype(jnp.bfloat16)
    ev9 = jnp.concatenate([ev_hi, ev_lo, ev_hi], axis=0)
    wt = W.T
    wt_hi = wt.astype(jnp.bfloat16)
    wt_lo = (wt - wt_hi.astype(jnp.float32)).astype(jnp.bfloat16)
    wt9 = jnp.concatenate([wt_hi, wt_hi, wt_lo], axis=1)
    d0, d1, d2 = _dedr_tc(ev9, wt9, wt, b.reshape(-1, 1))
    parts = _make_scatter(n_edges, aw)(
        d0, d1, d2, edge_index[0], edge_index[1])
    flat = _make_reduce(aw)(parts)
    return flat[: 3 * n_nodes].reshape(n_nodes, 3)


# speedup vs baseline: 1.4307x; 1.0612x over previous
import functools

import jax
import jax.numpy as jnp
from jax import lax
from jax.experimental import pallas as pl
from jax.experimental.pallas import tpu as pltpu
from jax.experimental.pallas import tpu_sc as plsc

_NC = 2
_NS = 16
_NW = _NC * _NS
_L = 16


def _dedr_body(ev9_ref, wt9_ref, wt_ref, b_ref, d0_ref, d1_ref, d2_ref):
    ev9 = ev9_ref[...]
    wt9 = wt9_ref[...]
    wt = wt_ref[...]
    h = b_ref[...] + lax.dot_general(
        wt9, ev9, dimension_numbers=(((1,), (0,)), ((), ())),
        precision=lax.Precision.DEFAULT,
        preferred_element_type=jnp.float32)
    t = jnp.tanh(h).astype(jnp.bfloat16)
    g = (1.0 - t * t).astype(jnp.bfloat16)
    d = lax.dot_general(
        wt, g, dimension_numbers=(((0,), (0,)), ((), ())),
        precision=lax.Precision.DEFAULT,
        preferred_element_type=jnp.float32)
    eb = ev9.shape[1]
    base = pl.program_id(0) * eb
    d0_ref[pl.ds(base, eb)] = d[0]
    d1_ref[pl.ds(base, eb)] = d[1]
    d2_ref[pl.ds(base, eb)] = d[2]


def _dedr_tc(ev9, Wt9, Wt, b2):
    n_edges = ev9.shape[1]
    h = Wt.shape[0]
    eb = 16000
    assert n_edges % eb == 0
    out1d = jax.ShapeDtypeStruct((n_edges,), jnp.float32)
    return pl.pallas_call(
        _dedr_body,
        grid=(n_edges // eb,),
        in_specs=[
            pl.BlockSpec((9, eb), lambda i: (0, i)),
            pl.BlockSpec((h, 9), lambda i: (0, 0)),
            pl.BlockSpec((h, 3), lambda i: (0, 0)),
            pl.BlockSpec((h, 1), lambda i: (0, 0)),
        ],
        out_specs=[pl.BlockSpec((n_edges,), lambda i: (0,))] * 3,
        out_shape=[out1d, out1d, out1d],
    )(ev9, Wt9, Wt, b2)


def _make_scatter(n_edges, aw):
    epw = n_edges // _NW
    assert n_edges % (_NW * _L) == 0 and epw % 8 == 0
    mesh = plsc.VectorSubcoreMesh(core_axis_name="c", subcore_axis_name="s")

    @functools.partial(
        pl.kernel,
        out_type=jax.ShapeDtypeStruct((_NW * aw,), jnp.float32),
        mesh=mesh,
        scratch_types=[
            pltpu.VMEM((epw,), jnp.float32),
            pltpu.VMEM((epw,), jnp.float32),
            pltpu.VMEM((epw,), jnp.float32),
            pltpu.VMEM((epw,), jnp.int32),
            pltpu.VMEM((epw,), jnp.int32),
            pltpu.VMEM((aw,), jnp.float32),
            pltpu.SemaphoreType.DMA,
        ],
        compiler_params=pltpu.CompilerParams(needs_layout_passes=False),
    )
    def scatter_k(d0_h, d1_h, d2_h, src_h, dst_h, out_h,
                  d0_v, d1_v, d2_v, src_v, dst_v, acc_v, sem):
        wid = lax.axis_index("s") * _NC + lax.axis_index("c")
        base = wid * epw
        d_v = (d0_v, d1_v, d2_v)
        cps = [pltpu.async_copy(h_ref.at[pl.ds(base, epw)], v_ref, sem)
               for h_ref, v_ref in ((d0_h, d0_v), (d1_h, d1_v), (d2_h, d2_v),
                                    (src_h, src_v), (dst_h, dst_v))]

        zeros = jnp.zeros((_L,), jnp.float32)
        zu = 16

        def zbody(i, carry):
            for u in range(zu):
                acc_v[pl.ds(i * (_L * zu) + u * _L, _L)] = zeros
            return carry

        lax.fori_loop(0, aw // (_L * zu), zbody, 0)
        for cp in cps:
            cp.wait()

        su = 25
        assert (epw // _L) % su == 0

        def sbody(g, carry):
            for u in range(su):
                off = g * (_L * su) + u * _L
                si = src_v[pl.ds(off, _L)] * 3
                di = dst_v[pl.ds(off, _L)] * 3
                for c in range(3):
                    v = d_v[c][pl.ds(off, _L)]
                    plsc.addupdate_scatter(acc_v, [si + c], v)
                    plsc.addupdate_scatter(acc_v, [di + c], -v)
            return carry

        lax.fori_loop(0, epw // (_L * su), sbody, 0)
        pltpu.sync_copy(acc_v, out_h.at[pl.ds(wid * aw, aw)])

    return scatter_k


def _make_reduce(aw):
    cw = aw // _NW
    assert cw % _L == 0 and cw % 8 == 0
    mesh = plsc.VectorSubcoreMesh(core_axis_name="c", subcore_axis_name="s")

    @functools.partial(
        pl.kernel,
        out_type=jax.ShapeDtypeStruct((aw,), jnp.float32),
        mesh=mesh,
        scratch_types=[
            pltpu.VMEM((_NW, cw), jnp.float32),
            pltpu.VMEM((cw,), jnp.float32),
            pltpu.SemaphoreType.DMA,
        ],
        compiler_params=pltpu.CompilerParams(needs_layout_passes=False),
    )
    def reduce_k(part_h, out_h, blk_v, res_v, sem):
        wid = lax.axis_index("s") * _NC + lax.axis_index("c")
        base = wid * cw
        cps = [pltpu.async_copy(part_h.at[pl.ds(r * aw + base, cw)],
                                blk_v.at[r], sem)
               for r in range(_NW)]
        for cp in cps:
            cp.wait()

        def cbody(j, carry):
            off = j * _L
            s = blk_v[0, pl.ds(off, _L)]
            for r in range(1, _NW):
                s = s + blk_v[r, pl.ds(off, _L)]
            res_v[pl.ds(off, _L)] = s
            return carry

        lax.fori_loop(0, cw // _L, cbody, 0)
        pltpu.sync_copy(res_v, out_h.at[pl.ds(base, cw)])

    return reduce_k


def kernel(node_feature, edge_vec, edge_index, W, b):
    n_nodes = node_feature.shape[0]
    n_edges = edge_vec.shape[0]
    aw = -(-(3 * n_nodes) // (_NW * 128)) * (_NW * 128)

    evT = edge_vec.T
    ev_hi = evT.astype(jnp.bfloat16)
    ev_lo = (evT - ev_hi.astype(jnp.float32)).astype(jnp.bfloat16)
    ev9 = jnp.concatenate([ev_hi, ev_lo, ev_hi], axis=0)
    wt = W.T
    wt_hi = wt.astype(jnp.bfloat16)
    wt_lo = (wt - wt_hi.astype(jnp.float32)).astype(jnp.bfloat16)
    wt9 = jnp.concatenate([wt_hi, wt_hi, wt_lo], axis=1)
    d0, d1, d2 = _dedr_tc(ev9, wt9, wt_hi, b.reshape(-1, 1))
    parts = _make_scatter(n_edges, aw)(
        d0, d1, d2, edge_index[0], edge_index[1])
    flat = _make_reduce(aw)(parts)
    return flat[: 3 * n_nodes].reshape(n_nodes, 3)


# speedup vs baseline: 1.4320x; 1.0009x over previous
import functools

import jax
import jax.numpy as jnp
from jax import lax
from jax.experimental import pallas as pl
from jax.experimental.pallas import tpu as pltpu
from jax.experimental.pallas import tpu_sc as plsc

_NC = 2
_NS = 16
_NW = _NC * _NS
_L = 16


def _dedr_body(ev9_ref, wt9_ref, wt_ref, b_ref, d0_ref, d1_ref, d2_ref):
    ev9 = ev9_ref[...]
    wt9 = wt9_ref[...]
    wt = wt_ref[...]
    h = b_ref[...] + lax.dot_general(
        wt9, ev9, dimension_numbers=(((1,), (0,)), ((), ())),
        precision=lax.Precision.DEFAULT,
        preferred_element_type=jnp.float32)
    t = jnp.tanh(h).astype(jnp.bfloat16)
    g = (1.0 - t * t).astype(jnp.bfloat16)
    d = lax.dot_general(
        wt, g, dimension_numbers=(((0,), (0,)), ((), ())),
        precision=lax.Precision.DEFAULT,
        preferred_element_type=jnp.float32)
    eb = ev9.shape[1]
    base = pl.program_id(0) * eb
    d0_ref[pl.ds(base, eb)] = d[0]
    d1_ref[pl.ds(base, eb)] = d[1]
    d2_ref[pl.ds(base, eb)] = d[2]


def _dedr_tc(ev9, Wt9, Wt, b2):
    n_edges = ev9.shape[1]
    h = Wt.shape[0]
    eb = 16000
    assert n_edges % eb == 0
    out1d = jax.ShapeDtypeStruct((n_edges,), jnp.float32)
    return pl.pallas_call(
        _dedr_body,
        grid=(n_edges // eb,),
        in_specs=[
            pl.BlockSpec((9, eb), lambda i: (0, i)),
            pl.BlockSpec((h, 9), lambda i: (0, 0)),
            pl.BlockSpec((h, 3), lambda i: (0, 0)),
            pl.BlockSpec((h, 1), lambda i: (0, 0)),
        ],
        out_specs=[pl.BlockSpec((n_edges,), lambda i: (0,))] * 3,
        out_shape=[out1d, out1d, out1d],
    )(ev9, Wt9, Wt, b2)


def _make_scatter(n_edges, aw):
    epw = n_edges // _NW
    assert n_edges % (_NW * _L) == 0 and epw % 8 == 0
    mesh = plsc.VectorSubcoreMesh(core_axis_name="c", subcore_axis_name="s")

    @functools.partial(
        pl.kernel,
        out_type=jax.ShapeDtypeStruct((_NW * aw,), jnp.float32),
        mesh=mesh,
        scratch_types=[
            pltpu.VMEM((epw,), jnp.float32),
            pltpu.VMEM((epw,), jnp.float32),
            pltpu.VMEM((epw,), jnp.float32),
            pltpu.VMEM((epw,), jnp.int32),
            pltpu.VMEM((epw,), jnp.int32),
            pltpu.VMEM((aw,), jnp.float32),
            pltpu.SemaphoreType.DMA,
        ],
        compiler_params=pltpu.CompilerParams(needs_layout_passes=False),
    )
    def scatter_k(d0_h, d1_h, d2_h, src_h, dst_h, out_h,
                  d0_v, d1_v, d2_v, src_v, dst_v, acc_v, sem):
        wid = lax.axis_index("s") * _NC + lax.axis_index("c")
        base = wid * epw
        d_v = (d0_v, d1_v, d2_v)
        cps = [pltpu.async_copy(h_ref.at[pl.ds(base, epw)], v_ref, sem)
               for h_ref, v_ref in ((d0_h, d0_v), (d1_h, d1_v), (d2_h, d2_v),
                                    (src_h, src_v), (dst_h, dst_v))]

        zeros = jnp.zeros((_L,), jnp.float32)
        zu = 16

        def zbody(i, carry):
            for u in range(zu):
                acc_v[pl.ds(i * (_L * zu) + u * _L, _L)] = zeros
            return carry

        lax.fori_loop(0, aw // (_L * zu), zbody, 0)
        for cp in cps:
            cp.wait()

        su = 25
        assert (epw // _L) % su == 0

        def sbody(g, carry):
            for u in range(su):
                off = g * (_L * su) + u * _L
                si = src_v[pl.ds(off, _L)] * 3
                di = dst_v[pl.ds(off, _L)] * 3
                for c in range(3):
                    v = d_v[c][pl.ds(off, _L)]
                    plsc.addupdate_scatter(acc_v, [si + c], v)
                    plsc.addupdate_scatter(acc_v, [di + c], -v)
            return carry

        lax.fori_loop(0, epw // (_L * su), sbody, 0)
        pltpu.sync_copy(acc_v, out_h.at[pl.ds(wid * aw, aw)])

    return scatter_k


def _make_reduce(aw):
    cw = aw // _NW
    assert cw % _L == 0 and cw % 8 == 0
    mesh = plsc.VectorSubcoreMesh(core_axis_name="c", subcore_axis_name="s")

    @functools.partial(
        pl.kernel,
        out_type=jax.ShapeDtypeStruct((aw,), jnp.float32),
        mesh=mesh,
        scratch_types=[
            pltpu.VMEM((_NW, cw), jnp.float32),
            pltpu.VMEM((cw,), jnp.float32),
            pltpu.SemaphoreType.DMA,
        ],
        compiler_params=pltpu.CompilerParams(needs_layout_passes=False),
    )
    def reduce_k(part_h, out_h, blk_v, res_v, sem):
        wid = lax.axis_index("s") * _NC + lax.axis_index("c")
        base = wid * cw
        cps = [pltpu.async_copy(part_h.at[pl.ds(r * aw + base, cw)],
                                blk_v.at[r], sem)
               for r in range(_NW)]
        for cp in cps:
            cp.wait()

        def cbody(j, carry):
            off = j * _L
            s = blk_v[0, pl.ds(off, _L)]
            for r in range(1, _NW):
                s = s + blk_v[r, pl.ds(off, _L)]
            res_v[pl.ds(off, _L)] = s
            return carry

        lax.fori_loop(0, cw // _L, cbody, 0)
        pltpu.sync_copy(res_v, out_h.at[pl.ds(base, cw)])

    return reduce_k


def kernel(node_feature, edge_vec, edge_index, W, b):
    n_nodes = node_feature.shape[0]
    n_edges = edge_vec.shape[0]
    aw = -(-(3 * n_nodes) // (_NW * 128)) * (_NW * 128)

    ev_hi = edge_vec.astype(jnp.bfloat16)
    ev_lo = (edge_vec - ev_hi.astype(jnp.float32)).astype(jnp.bfloat16)
    ev9 = jnp.concatenate([ev_hi, ev_lo, ev_hi], axis=1).T
    wt = W.T
    wt_hi = wt.astype(jnp.bfloat16)
    wt_lo = (wt - wt_hi.astype(jnp.float32)).astype(jnp.bfloat16)
    wt9 = jnp.concatenate([wt_hi, wt_hi, wt_lo], axis=1)
    d0, d1, d2 = _dedr_tc(ev9, wt9, wt_hi, b.reshape(-1, 1))
    parts = _make_scatter(n_edges, aw)(
        d0, d1, d2, edge_index[0], edge_index[1])
    flat = _make_reduce(aw)(parts)
    return flat[: 3 * n_nodes].reshape(n_nodes, 3)


# speedup vs baseline: 1.4416x; 1.0066x over previous
import functools

import jax
import jax.numpy as jnp
from jax import lax
from jax.experimental import pallas as pl
from jax.experimental.pallas import tpu as pltpu
from jax.experimental.pallas import tpu_sc as plsc

_NC = 2
_NS = 16
_NW = _NC * _NS
_L = 16


def _dedr_body(ev9_ref, wt9_ref, wt_ref, b_ref, d0_ref, d1_ref, d2_ref):
    ev9 = ev9_ref[...]
    wt9 = wt9_ref[...]
    wt = wt_ref[...]
    h = b_ref[...] + lax.dot_general(
        wt9, ev9, dimension_numbers=(((1,), (0,)), ((), ())),
        precision=lax.Precision.DEFAULT,
        preferred_element_type=jnp.float32)
    t = jnp.tanh(h).astype(jnp.bfloat16)
    g = (1.0 - t * t).astype(jnp.bfloat16)
    d = lax.dot_general(
        wt, g, dimension_numbers=(((0,), (0,)), ((), ())),
        precision=lax.Precision.DEFAULT,
        preferred_element_type=jnp.float32)
    eb = ev9.shape[1]
    base = pl.program_id(0) * eb
    d0_ref[pl.ds(base, eb)] = d[0]
    d1_ref[pl.ds(base, eb)] = d[1]
    d2_ref[pl.ds(base, eb)] = d[2]


def _dedr_tc(ev9, Wt9, Wt, b2, half, lo, nh):
    h = Wt.shape[0]
    eb = 16000
    assert nh % eb == 0 and lo % eb == 0
    off = lo // eb
    out1d = jax.ShapeDtypeStruct((nh,), jnp.float32)
    return pl.pallas_call(
        _dedr_body,
        grid=(nh // eb,),
        in_specs=[
            pl.BlockSpec((9, eb), lambda i: (0, i + off)),
            pl.BlockSpec((h, 9), lambda i: (0, 0)),
            pl.BlockSpec((h, 3), lambda i: (0, 0)),
            pl.BlockSpec((h, 1), lambda i: (0, 0)),
        ],
        out_specs=[pl.BlockSpec((nh,), lambda i: (0,))] * 3,
        out_shape=[out1d, out1d, out1d],
    )(ev9, Wt9, Wt, b2)


def _make_scatter(n_edges, aw):
    epw = n_edges // _NW
    assert n_edges % (_NW * _L) == 0 and epw % 8 == 0
    mesh = plsc.VectorSubcoreMesh(core_axis_name="c", subcore_axis_name="s")

    @functools.partial(
        pl.kernel,
        out_type=jax.ShapeDtypeStruct((_NW * aw,), jnp.float32),
        mesh=mesh,
        scratch_types=[
            pltpu.VMEM((epw,), jnp.float32),
            pltpu.VMEM((epw,), jnp.float32),
            pltpu.VMEM((epw,), jnp.float32),
            pltpu.VMEM((epw,), jnp.int32),
            pltpu.VMEM((epw,), jnp.int32),
            pltpu.VMEM((aw,), jnp.float32),
            pltpu.SemaphoreType.DMA,
        ],
        compiler_params=pltpu.CompilerParams(needs_layout_passes=False),
    )
    def scatter_k(d0_h, d1_h, d2_h, src_h, dst_h, out_h,
                  d0_v, d1_v, d2_v, src_v, dst_v, acc_v, sem):
        wid = lax.axis_index("s") * _NC + lax.axis_index("c")
        base = wid * epw
        d_v = (d0_v, d1_v, d2_v)
        cps = [pltpu.async_copy(h_ref.at[pl.ds(base, epw)], v_ref, sem)
               for h_ref, v_ref in ((d0_h, d0_v), (d1_h, d1_v), (d2_h, d2_v),
                                    (src_h, src_v), (dst_h, dst_v))]

        zeros = jnp.zeros((_L,), jnp.float32)
        zu = 16

        def zbody(i, carry):
            for u in range(zu):
                acc_v[pl.ds(i * (_L * zu) + u * _L, _L)] = zeros
            return carry

        lax.fori_loop(0, aw // (_L * zu), zbody, 0)
        for cp in cps:
            cp.wait()

        su = 25
        assert (epw // _L) % su == 0

        def sbody(g, carry):
            for u in range(su):
                off = g * (_L * su) + u * _L
                si = src_v[pl.ds(off, _L)] * 3
                di = dst_v[pl.ds(off, _L)] * 3
                for c in range(3):
                    v = d_v[c][pl.ds(off, _L)]
                    plsc.addupdate_scatter(acc_v, [si + c], v)
                    plsc.addupdate_scatter(acc_v, [di + c], -v)
            return carry

        lax.fori_loop(0, epw // (_L * su), sbody, 0)
        pltpu.sync_copy(acc_v, out_h.at[pl.ds(wid * aw, aw)])

    return scatter_k


def _make_reduce(aw):
    cw = aw // _NW
    assert cw % _L == 0 and cw % 8 == 0
    mesh = plsc.VectorSubcoreMesh(core_axis_name="c", subcore_axis_name="s")

    @functools.partial(
        pl.kernel,
        out_type=jax.ShapeDtypeStruct((aw,), jnp.float32),
        mesh=mesh,
        scratch_types=[
            pltpu.VMEM((2 * _NW, cw), jnp.float32),
            pltpu.VMEM((cw,), jnp.float32),
            pltpu.SemaphoreType.DMA,
        ],
        compiler_params=pltpu.CompilerParams(needs_layout_passes=False),
    )
    def reduce_k(p0_h, p1_h, out_h, blk_v, res_v, sem):
        wid = lax.axis_index("s") * _NC + lax.axis_index("c")
        base = wid * cw
        cps = [pltpu.async_copy(p_h.at[pl.ds(r * aw + base, cw)],
                                blk_v.at[k * _NW + r], sem)
               for k, p_h in enumerate((p0_h, p1_h))
               for r in range(_NW)]
        for cp in cps:
            cp.wait()

        def cbody(j, carry):
            off = j * _L
            s = blk_v[0, pl.ds(off, _L)]
            for r in range(1, 2 * _NW):
                s = s + blk_v[r, pl.ds(off, _L)]
            res_v[pl.ds(off, _L)] = s
            return carry

        lax.fori_loop(0, cw // _L, cbody, 0)
        pltpu.sync_copy(res_v, out_h.at[pl.ds(base, cw)])

    return reduce_k


def kernel(node_feature, edge_vec, edge_index, W, b):
    n_nodes = node_feature.shape[0]
    n_edges = edge_vec.shape[0]
    aw = -(-(3 * n_nodes) // (_NW * 128)) * (_NW * 128)

    ev_hi = edge_vec.astype(jnp.bfloat16)
    ev_lo = (edge_vec - ev_hi.astype(jnp.float32)).astype(jnp.bfloat16)
    ev9 = jnp.concatenate([ev_hi, ev_lo, ev_hi], axis=1).T
    wt = W.T
    wt_hi = wt.astype(jnp.bfloat16)
    wt_lo = (wt - wt_hi.astype(jnp.float32)).astype(jnp.bfloat16)
    wt9 = jnp.concatenate([wt_hi, wt_hi, wt_lo], axis=1)
    b2 = b.reshape(-1, 1)
    na = n_edges * 3 // 5
    parts = []
    for half, (lo, sz) in enumerate(((0, na), (na, n_edges - na))):
        d0, d1, d2 = _dedr_tc(ev9, wt9, wt_hi, b2, half, lo, sz)
        parts.append(_make_scatter(sz, aw)(
            d0, d1, d2,
            lax.slice(edge_index, (0, lo), (1, lo + sz)).reshape(sz),
            lax.slice(edge_index, (1, lo), (2, lo + sz)).reshape(sz)))
    flat = _make_reduce(aw)(parts[0], parts[1])
    return flat[: 3 * n_nodes].reshape(n_nodes, 3)


# speedup vs baseline: 1.4416x; 1.0000x over previous
import functools

import jax
import jax.numpy as jnp
from jax import lax
from jax.experimental import pallas as pl
from jax.experimental.pallas import tpu as pltpu
from jax.experimental.pallas import tpu_sc as plsc

_NC = 2
_NS = 16
_NW = _NC * _NS
_L = 16


def _dedr_body(ev9_ref, wt9_ref, wt_ref, b_ref, d0_ref, d1_ref, d2_ref):
    ev9 = ev9_ref[...]
    wt9 = wt9_ref[...]
    wt = wt_ref[...]
    h = b_ref[...] + lax.dot_general(
        wt9, ev9, dimension_numbers=(((1,), (0,)), ((), ())),
        precision=lax.Precision.DEFAULT,
        preferred_element_type=jnp.float32)
    t = jnp.tanh(h).astype(jnp.bfloat16)
    g = (1.0 - t * t).astype(jnp.bfloat16)
    d = lax.dot_general(
        wt, g, dimension_numbers=(((0,), (0,)), ((), ())),
        precision=lax.Precision.DEFAULT,
        preferred_element_type=jnp.float32)
    eb = ev9.shape[1]
    base = pl.program_id(0) * eb
    d0_ref[pl.ds(base, eb)] = d[0]
    d1_ref[pl.ds(base, eb)] = d[1]
    d2_ref[pl.ds(base, eb)] = d[2]


def _dedr_tc(ev9, Wt9, Wt, b2, lo, nh):
    h = Wt.shape[0]
    eb = 16000
    assert nh % eb == 0 and lo % eb == 0
    off = lo // eb
    out1d = jax.ShapeDtypeStruct((nh,), jnp.float32)
    return pl.pallas_call(
        _dedr_body,
        grid=(nh // eb,),
        in_specs=[
            pl.BlockSpec((9, eb), lambda i: (0, i + off)),
            pl.BlockSpec((h, 9), lambda i: (0, 0)),
            pl.BlockSpec((h, 3), lambda i: (0, 0)),
            pl.BlockSpec((h, 1), lambda i: (0, 0)),
        ],
        out_specs=[pl.BlockSpec((nh,), lambda i: (0,))] * 3,
        out_shape=[out1d, out1d, out1d],
    )(ev9, Wt9, Wt, b2)


def _make_scatter(n_edges, aw):
    epw = n_edges // _NW
    assert n_edges % (_NW * _L) == 0 and epw % 8 == 0
    mesh = plsc.VectorSubcoreMesh(core_axis_name="c", subcore_axis_name="s")

    @functools.partial(
        pl.kernel,
        out_type=jax.ShapeDtypeStruct((_NW * aw,), jnp.float32),
        mesh=mesh,
        scratch_types=[
            pltpu.VMEM((epw,), jnp.float32),
            pltpu.VMEM((epw,), jnp.float32),
            pltpu.VMEM((epw,), jnp.float32),
            pltpu.VMEM((epw,), jnp.int32),
            pltpu.VMEM((epw,), jnp.int32),
            pltpu.VMEM((aw,), jnp.float32),
            pltpu.SemaphoreType.DMA,
        ],
        compiler_params=pltpu.CompilerParams(needs_layout_passes=False),
    )
    def scatter_k(d0_h, d1_h, d2_h, src_h, dst_h, out_h,
                  d0_v, d1_v, d2_v, src_v, dst_v, acc_v, sem):
        wid = lax.axis_index("s") * _NC + lax.axis_index("c")
        base = wid * epw
        d_v = (d0_v, d1_v, d2_v)
        cps = [pltpu.async_copy(h_ref.at[pl.ds(base, epw)], v_ref, sem)
               for h_ref, v_ref in ((d0_h, d0_v), (d1_h, d1_v), (d2_h, d2_v),
                                    (src_h, src_v), (dst_h, dst_v))]

        zeros = jnp.zeros((_L,), jnp.float32)
        zu = 16

        def zbody(i, carry):
            for u in range(zu):
                acc_v[pl.ds(i * (_L * zu) + u * _L, _L)] = zeros
            return carry

        lax.fori_loop(0, aw // (_L * zu), zbody, 0)
        for cp in cps:
            cp.wait()

        su = 25
        assert (epw // _L) % su == 0

        def sbody(g, carry):
            for u in range(su):
                off = g * (_L * su) + u * _L
                si = src_v[pl.ds(off, _L)] * 3
                di = dst_v[pl.ds(off, _L)] * 3
                for c in range(3):
                    v = d_v[c][pl.ds(off, _L)]
                    plsc.addupdate_scatter(acc_v, [si + c], v)
                    plsc.addupdate_scatter(acc_v, [di + c], -v)
            return carry

        lax.fori_loop(0, epw // (_L * su), sbody, 0)
        pltpu.sync_copy(acc_v, out_h.at[pl.ds(wid * aw, aw)])

    return scatter_k


def _make_reduce(aw):
    cw = aw // _NW
    assert cw % _L == 0 and cw % 8 == 0
    mesh = plsc.VectorSubcoreMesh(core_axis_name="c", subcore_axis_name="s")

    @functools.partial(
        pl.kernel,
        out_type=jax.ShapeDtypeStruct((aw,), jnp.float32),
        mesh=mesh,
        scratch_types=[
            pltpu.VMEM((2 * _NW, cw), jnp.float32),
            pltpu.VMEM((cw,), jnp.float32),
            pltpu.SemaphoreType.DMA,
        ],
        compiler_params=pltpu.CompilerParams(needs_layout_passes=False),
    )
    def reduce_k(p0_h, p1_h, out_h, blk_v, res_v, sem):
        wid = lax.axis_index("s") * _NC + lax.axis_index("c")
        base = wid * cw
        cps = [pltpu.async_copy(p_h.at[pl.ds(r * aw + base, cw)],
                                blk_v.at[k * _NW + r], sem)
               for k, p_h in enumerate((p0_h, p1_h))
               for r in range(_NW)]
        for cp in cps:
            cp.wait()

        def cbody(j, carry):
            off = j * _L
            s = blk_v[0, pl.ds(off, _L)]
            for r in range(1, 2 * _NW):
                s = s + blk_v[r, pl.ds(off, _L)]
            res_v[pl.ds(off, _L)] = s
            return carry

        lax.fori_loop(0, cw // _L, cbody, 0)
        pltpu.sync_copy(res_v, out_h.at[pl.ds(base, cw)])

    return reduce_k


def kernel(node_feature, edge_vec, edge_index, W, b):
    n_nodes = node_feature.shape[0]
    n_edges = edge_vec.shape[0]
    aw = -(-(3 * n_nodes) // (_NW * 128)) * (_NW * 128)

    ev_hi = edge_vec.astype(jnp.bfloat16)
    ev_lo = (edge_vec - ev_hi.astype(jnp.float32)).astype(jnp.bfloat16)
    ev9 = jnp.concatenate([ev_hi, ev_lo, ev_hi], axis=1).T
    wt = W.T
    wt_hi = wt.astype(jnp.bfloat16)
    wt_lo = (wt - wt_hi.astype(jnp.float32)).astype(jnp.bfloat16)
    wt9 = jnp.concatenate([wt_hi, wt_hi, wt_lo], axis=1)
    b2 = b.reshape(-1, 1)
    na = n_edges * 3 // 5
    parts = []
    for lo, sz in ((0, na), (na, n_edges - na)):
        d0, d1, d2 = _dedr_tc(ev9, wt9, wt_hi, b2, lo, sz)
        parts.append(_make_scatter(sz, aw)(
            d0, d1, d2,
            lax.slice(edge_index, (0, lo), (1, lo + sz)).reshape(sz),
            lax.slice(edge_index, (1, lo), (2, lo + sz)).reshape(sz)))
    flat = _make_reduce(aw)(parts[0], parts[1])
    return flat[: 3 * n_nodes].reshape(n_nodes, 3)
